# baseline (speedup 1.0000x reference)
import functools

import jax
import jax.numpy as jnp
from jax import lax
from jax.experimental import pallas as pl
from jax.experimental.pallas import tpu as pltpu
from jax.experimental.pallas import tpu_sc as plsc

_B = 16384
_D = 64
_NC = 2
_NS = 16
_NW = _NC * _NS
_BPW = _B // _NW
_L = 16


def _tt_kernel(user_idx, item_idx, ut, it, out_hbm,
               uidx_v, iidx_v, gu_v, gi_v, out_v, sem_u, sem_i):
    wid = lax.axis_index("s") * _NC + lax.axis_index("c")
    lane_iota = lax.iota(jnp.int32, _L)
    pltpu.sync_copy(user_idx.at[pl.ds(wid * _BPW, _BPW)], uidx_v)
    pltpu.sync_copy(item_idx.at[pl.ds(wid * _BPW, _BPW)], iidx_v)

    def chunk(h, carry):
        uvec = uidx_v[pl.ds(h * _L, _L)]
        ivec = iidx_v[pl.ds(h * _L, _L)]
        ug = (uvec >> 3) << 3
        ig = (ivec >> 3) << 3
        ur = uvec & 7
        ir = ivec & 7

        copies = []
        urs, irs = [], []
        for j in range(_L):
            sel = lane_iota == j
            sug = pl.multiple_of(jnp.sum(jnp.where(sel, ug, 0)), 8)
            sig = pl.multiple_of(jnp.sum(jnp.where(sel, ig, 0)), 8)
            urs.append(jnp.sum(jnp.where(sel, ur, 0)))
            irs.append(jnp.sum(jnp.where(sel, ir, 0)))
            copies.append(pltpu.async_copy(
                ut.at[pl.ds(sug, 8), :], gu_v.at[j], sem_u))
            copies.append(pltpu.async_copy(
                it.at[pl.ds(sig, 8), :], gi_v.at[j], sem_i))
        for c in copies:
            c.wait()

        acc = jnp.zeros((_L,), jnp.float32)
        for j in range(_L):
            su, si = urs[j], irs[j]
            pu = (gu_v[j, su, pl.ds(0, _L)] * gi_v[j, si, pl.ds(0, _L)]
                  + gu_v[j, su, pl.ds(_L, _L)] * gi_v[j, si, pl.ds(_L, _L)]
                  + gu_v[j, su, pl.ds(2 * _L, _L)]
                  * gi_v[j, si, pl.ds(2 * _L, _L)]
                  + gu_v[j, su, pl.ds(3 * _L, _L)]
                  * gi_v[j, si, pl.ds(3 * _L, _L)])
            acc = jnp.where(lane_iota == j, jnp.sum(pu), acc)
        out_v[pl.ds(h * _L, _L)] = acc
        return carry

    lax.fori_loop(0, _BPW // _L, chunk, 0)

    pltpu.sync_copy(out_v, out_hbm.at[pl.ds(wid * _BPW, _BPW)])


@jax.jit
def kernel(user_idx, item_idx, user_table, item_table):
    mesh = plsc.VectorSubcoreMesh(core_axis_name="c", subcore_axis_name="s")
    f = functools.partial(
        pl.kernel,
        out_type=jax.ShapeDtypeStruct((_B,), jnp.float32),
        mesh=mesh,
        compiler_params=pltpu.CompilerParams(needs_layout_passes=False),
        scratch_types=[
            pltpu.VMEM((_BPW,), jnp.int32),
            pltpu.VMEM((_BPW,), jnp.int32),
            pltpu.VMEM((_L, 8, _D), jnp.float32),
            pltpu.VMEM((_L, 8, _D), jnp.float32),
            pltpu.VMEM((_BPW,), jnp.float32),
            pltpu.SemaphoreType.DMA,
            pltpu.SemaphoreType.DMA,
        ],
    )(_tt_kernel)
    return f(user_idx.astype(jnp.int32), item_idx.astype(jnp.int32),
             user_table, item_table)
